# Initial kernel scaffold; baseline (speedup 1.0000x reference)
import jax
import jax.numpy as jnp
from jax.experimental import pallas as pl


def kernel(x, edge_index, child_c, child_h, W_ioux, b_ioux, W_iouh, b_iouh, W_fx, b_fx, W_fh, b_fh):
    raise NotImplementedError("write your pallas kernel here")


# speedup vs baseline: 5.8489x; 5.8489x over previous
import functools

import jax
import jax.numpy as jnp
from jax import lax
from jax.experimental import pallas as pl
from jax.experimental.pallas import tpu as pltpu
from jax.experimental.pallas import tpu_sc as plsc

N = 50000
E = 800000
D_IN = 300
H = 64

NC = 2
NS = 16
LANES = 16

ROUNDS = 4
NPART = NC * ROUNDS
PSIZE = 6256
ACC_ROWS = PSIZE + LANES
OUT_ROWS = NPART * PSIZE
ETILE = E // NS
CHUNK = 2000
NCHUNK = ETILE // CHUNK
NGRP = CHUNK // LANES
SUB = 96
NZ = ACC_ROWS // NS
OW = 400
OW_LAST = PSIZE - (NS - 1) * OW


def _sc_mesh():
  return plsc.VectorSubcoreMesh(core_axis_name="c", subcore_axis_name="s")


@functools.partial(
    pl.kernel,
    out_type=jax.ShapeDtypeStruct((OUT_ROWS, 2 * H), jnp.float32),
    mesh=_sc_mesh(),
    scratch_types=dict(
        sbuf=pltpu.VMEM((CHUNK,), jnp.int32),
        dbuf=pltpu.VMEM((CHUNK,), jnp.int32),
        csrc=pltpu.VMEM((CHUNK + 2 * SUB,), jnp.int32),
        cdst=pltpu.VMEM((CHUNK + 2 * SUB,), jnp.int32),
        idxs=pltpu.VMEM((SUB,), jnp.int32),
        idxd=pltpu.VMEM((SUB,), jnp.int32),
        idxg=pltpu.VMEM((SUB,), jnp.int32),
        g1=pltpu.VMEM((SUB, 2 * H), jnp.float32),
        g2a=pltpu.VMEM((SUB, 2 * H), jnp.float32),
        g2b=pltpu.VMEM((SUB, 2 * H), jnp.float32),
        obuf=pltpu.VMEM((SUB, 2 * H), jnp.float32),
        acc=pltpu.VMEM_SHARED((ACC_ROWS, 2 * H), jnp.float32),
        sem=pltpu.SemaphoreType.DMA,
    ),
    compiler_params=pltpu.CompilerParams(needs_layout_passes=False),
)
def _sc_edges(src_hbm, dst_hbm, thc_hbm, tf_hbm, zeros_hbm, out_hbm,
              sbuf, dbuf, csrc, cdst, idxs, idxd, idxg, g1, g2a, g2b, obuf,
              acc, sem):
  cid = lax.axis_index("c")
  sid = lax.axis_index("s")
  base_e = sid * ETILE
  iota = lax.iota(jnp.int32, LANES)

  def flush(nfull, off, lo):
    def sub(j, _):
      jb = j * SUB
      for i in range(SUB // LANES):
        sl = pl.ds(i * LANES, LANES)
        vs = csrc[pl.ds(jb + i * LANES, LANES)]
        vd = cdst[pl.ds(jb + i * LANES, LANES)]
        idxs[sl] = vs
        idxd[sl] = vd
        idxg[sl] = jnp.minimum(vd + lo, N - 1)
      c1 = pltpu.async_copy(thc_hbm.at[idxs], g1, sem)
      c2 = pltpu.async_copy(tf_hbm.at[idxs], g2a, sem)
      c3 = pltpu.async_copy(tf_hbm.at[idxg], g2b, sem)
      c1.wait()
      c2.wait()
      c3.wait()

      def row(r0, _):
        for q in range(H // LANES):
          slh = pl.ds(q * LANES, LANES)
          slc = pl.ds(H + q * LANES, LANES)
          obuf[r0, slh] = g1[r0, slh]
          z = g2a[r0, slh] + g2b[r0, slc]
          f = 1.0 / (1.0 + jnp.exp(-z))
          obuf[r0, slc] = f * g1[r0, slc]
        return 0

      lax.fori_loop(0, SUB, row, 0)
      pltpu.sync_copy(obuf, acc.at[idxd], add=True)
      return 0

    lax.fori_loop(0, nfull, sub, 0)
    rem_base = nfull * SUB
    for i in range(SUB // LANES):
      v1 = csrc[pl.ds(rem_base + i * LANES, LANES)]
      v2 = cdst[pl.ds(rem_base + i * LANES, LANES)]
      csrc[pl.ds(i * LANES, LANES)] = v1
      cdst[pl.ds(i * LANES, LANES)] = v2
    return off - rem_base

  def round_body(rr, _):
    p = cid * ROUNDS + rr
    lo = p * PSIZE
    pltpu.sync_copy(zeros_hbm.at[pl.ds(sid * NZ, NZ)],
                    acc.at[pl.ds(sid * NZ, NZ)])
    plsc.subcore_barrier()

    def chunk_body(ch, off):
      off0 = pl.multiple_of(base_e + ch * CHUNK, 8)
      pltpu.sync_copy(src_hbm.at[pl.ds(off0, CHUNK)], sbuf)
      pltpu.sync_copy(dst_hbm.at[pl.ds(off0, CHUNK)], dbuf)

      def grp(g, off):
        gb = g * LANES
        s16 = sbuf[pl.ds(gb, LANES)]
        dl = dbuf[pl.ds(gb, LANES)] - lo
        m = (dl >= 0) & (dl < PSIZE)
        cnt = jnp.sum(m.astype(jnp.int32))
        plsc.store_compressed(csrc.at[pl.ds(off, LANES)], s16, mask=m)
        plsc.store_compressed(cdst.at[pl.ds(off, LANES)], dl, mask=m)
        return off + cnt

      off = lax.fori_loop(0, NGRP, grp, off)
      return flush(off // SUB, off, lo)

    off = lax.fori_loop(0, NCHUNK, chunk_body, 0)
    for i in range(SUB // LANES):
      csrc[pl.ds(off + i * LANES, LANES)] = iota + i * LANES
      cdst[pl.ds(off + i * LANES, LANES)] = iota + PSIZE
    flush((off + SUB - 1) // SUB, 0, lo)

    plsc.subcore_barrier()
    base_o = pl.multiple_of(p * PSIZE, 8)

    @pl.when(sid < NS - 1)
    def _():
      o = pl.multiple_of(sid * OW, 8)
      pltpu.sync_copy(acc.at[pl.ds(o, OW)], out_hbm.at[pl.ds(base_o + o, OW)])

    @pl.when(sid == NS - 1)
    def _():
      o = (NS - 1) * OW
      pltpu.sync_copy(acc.at[pl.ds(o, OW_LAST)],
                      out_hbm.at[pl.ds(base_o + o, OW_LAST)])

    plsc.subcore_barrier()
    return 0

  lax.fori_loop(0, ROUNDS, round_body, 0)


_ROWB = 2000
_GRID = N // _ROWB


def _pre_body(x_ref, cc_ref, ch_ref, wioux_ref, bsum_ref, wfx_ref, bfx_ref,
              wfh_ref, bfh_ref, xw_ref, thc_ref, tf_ref):
  x = x_ref[...]
  ch = ch_ref[...]
  xw_ref[...] = (
      jnp.dot(x, wioux_ref[...], preferred_element_type=jnp.float32)
      + bsum_ref[...])
  thc_ref[...] = jnp.concatenate([ch, cc_ref[...]], axis=1)
  fh = jnp.dot(ch, wfh_ref[...], preferred_element_type=jnp.float32) + bfh_ref[...]
  fx = jnp.dot(x, wfx_ref[...], preferred_element_type=jnp.float32) + bfx_ref[...]
  tf_ref[...] = jnp.concatenate([fh, fx], axis=1)


def _pre(x, child_c, child_h, W_ioux, bsum, W_fx, bfx, W_fh, bfh):
  return pl.pallas_call(
      _pre_body,
      grid=(_GRID,),
      in_specs=[
          pl.BlockSpec((_ROWB, D_IN), lambda i: (i, 0)),
          pl.BlockSpec((_ROWB, H), lambda i: (i, 0)),
          pl.BlockSpec((_ROWB, H), lambda i: (i, 0)),
          pl.BlockSpec((D_IN, 3 * H), lambda i: (0, 0)),
          pl.BlockSpec((1, 3 * H), lambda i: (0, 0)),
          pl.BlockSpec((D_IN, H), lambda i: (0, 0)),
          pl.BlockSpec((1, H), lambda i: (0, 0)),
          pl.BlockSpec((H, H), lambda i: (0, 0)),
          pl.BlockSpec((1, H), lambda i: (0, 0)),
      ],
      out_specs=[
          pl.BlockSpec((_ROWB, 3 * H), lambda i: (i, 0)),
          pl.BlockSpec((_ROWB, 2 * H), lambda i: (i, 0)),
          pl.BlockSpec((_ROWB, 2 * H), lambda i: (i, 0)),
      ],
      out_shape=[
          jax.ShapeDtypeStruct((N, 3 * H), jnp.float32),
          jax.ShapeDtypeStruct((N, 2 * H), jnp.float32),
          jax.ShapeDtypeStruct((N, 2 * H), jnp.float32),
      ],
  )(x, child_c, child_h, W_ioux, bsum, W_fx, bfx, W_fh, bfh)


def _post_body(hf_ref, xw_ref, wiouh_ref, c_ref, h_ref):
  hf = hf_ref[...]
  iou = xw_ref[...] + jnp.dot(
      hf[:, 0:H], wiouh_ref[...], preferred_element_type=jnp.float32)
  i = jax.nn.sigmoid(iou[:, 0:H])
  o = jax.nn.sigmoid(iou[:, H:2 * H])
  u = jnp.tanh(iou[:, 2 * H:3 * H])
  c = i * u + hf[:, H:2 * H]
  c_ref[...] = c
  h_ref[...] = o * jnp.tanh(c)


def _post(hf, xw, W_iouh):
  return pl.pallas_call(
      _post_body,
      grid=(_GRID,),
      in_specs=[
          pl.BlockSpec((_ROWB, 2 * H), lambda i: (i, 0)),
          pl.BlockSpec((_ROWB, 3 * H), lambda i: (i, 0)),
          pl.BlockSpec((H, 3 * H), lambda i: (0, 0)),
      ],
      out_specs=[
          pl.BlockSpec((_ROWB, H), lambda i: (i, 0)),
          pl.BlockSpec((_ROWB, H), lambda i: (i, 0)),
      ],
      out_shape=[
          jax.ShapeDtypeStruct((N, H), jnp.float32),
          jax.ShapeDtypeStruct((N, H), jnp.float32),
      ],
  )(hf, xw, W_iouh)


def kernel(x, edge_index, child_c, child_h,
           W_ioux, b_ioux, W_iouh, b_iouh, W_fx, b_fx, W_fh, b_fh):
  src = edge_index[0]
  dst = edge_index[1]
  bsum = (b_ioux + b_iouh).reshape(1, 3 * H)
  xw, thc, tf = _pre(x, child_c, child_h, W_ioux, bsum,
                     W_fx, b_fx.reshape(1, H), W_fh, b_fh.reshape(1, H))
  zeros_acc = jnp.zeros((ACC_ROWS, 2 * H), jnp.float32)
  hf = _sc_edges(src, dst, thc, tf, zeros_acc)
  c, h = _post(hf[:N], xw, W_iouh)
  return (c, h)
